# Initial kernel scaffold; baseline (speedup 1.0000x reference)
import functools

import jax
import jax.numpy as jnp
from jax import lax
from jax.experimental import pallas as pl
from jax.experimental.pallas import tpu as pltpu
from jax.experimental.pallas import tpu_sc as plsc

N = 10000
E = 320000
D = 128

NC = 2
NS = 16
NW = NC * NS
EPC = E // NC
EPT = E // NW
RPT = N // NS
K = 80
ZR = 125

_mesh = plsc.VectorSubcoreMesh(core_axis_name="core", subcore_axis_name="subcore")


def _fill(ref, rows, value):
    cols = ref.shape[1]
    vec = jnp.full((16,), value, dtype=ref.dtype)

    @pl.loop(0, rows)
    def _(i):
        @pl.loop(0, cols, step=16)
        def _(j):
            ref[i, pl.ds(j, 16)] = vec


@functools.partial(
    pl.kernel,
    out_type=jax.ShapeDtypeStruct((NC, 2, N, 16), jnp.float32),
    mesh=_mesh,
    scratch_types=[
        pltpu.VMEM_SHARED((N, 16), jnp.float32),
        pltpu.VMEM_SHARED((N, 16), jnp.float32),
        pltpu.VMEM((K, 16), jnp.float32),
        pltpu.VMEM((RPT, 16), jnp.float32),
        pltpu.VMEM((K,), jnp.int32),
        pltpu.VMEM((K,), jnp.int32),
    ],
)
def _deg_kernel(src_hbm, dst_hbm, deg_hbm, dsrc_sh, ddst_sh, ones_v, zeros_v,
                sidx, didx):
    c = lax.axis_index("core")
    s = lax.axis_index("subcore")
    _fill(ones_v, K, 1.0)
    _fill(zeros_v, RPT, 0.0)
    pltpu.sync_copy(zeros_v, dsrc_sh.at[pl.ds(s * RPT, RPT)])
    pltpu.sync_copy(zeros_v, ddst_sh.at[pl.ds(s * RPT, RPT)])
    plsc.subcore_barrier()

    ebase = c * EPC + s * EPT

    @pl.loop(0, EPT // K)
    def _(j):
        base = ebase + j * K
        pltpu.sync_copy(src_hbm.at[pl.ds(base, K)], sidx)
        pltpu.sync_copy(dst_hbm.at[pl.ds(base, K)], didx)
        pltpu.sync_copy(ones_v, dsrc_sh.at[sidx], add=True)
        pltpu.sync_copy(ones_v, ddst_sh.at[didx], add=True)

    plsc.subcore_barrier()
    rows = pl.ds(s * RPT, RPT)
    pltpu.sync_copy(dsrc_sh.at[rows], deg_hbm.at[c, 0, rows])
    pltpu.sync_copy(ddst_sh.at[rows], deg_hbm.at[c, 1, rows])


@functools.partial(
    pl.kernel,
    out_type=jax.ShapeDtypeStruct((NC, N, D), jnp.float32),
    mesh=_mesh,
    scratch_types=[
        pltpu.VMEM_SHARED((N, D), jnp.float32),
        pltpu.VMEM((ZR, D), jnp.float32),
        pltpu.VMEM((K, D), jnp.float32),
        pltpu.VMEM((K,), jnp.int32),
        pltpu.VMEM((K,), jnp.int32),
    ],
)
def _agg_kernel(h_hbm, src_hbm, dst_hbm, pagg_hbm, agg_sh, zeros_v, rows_v,
                sidx, didx):
    c = lax.axis_index("core")
    s = lax.axis_index("subcore")
    _fill(zeros_v, ZR, 0.0)

    @pl.loop(0, RPT // ZR)
    def _(z):
        pltpu.sync_copy(zeros_v, agg_sh.at[pl.ds(s * RPT + z * ZR, ZR)])

    plsc.subcore_barrier()

    ebase = c * EPC + s * EPT

    @pl.loop(0, EPT // K)
    def _(j):
        base = ebase + j * K
        pltpu.sync_copy(src_hbm.at[pl.ds(base, K)], sidx)
        pltpu.sync_copy(dst_hbm.at[pl.ds(base, K)], didx)
        pltpu.sync_copy(h_hbm.at[sidx], rows_v)
        pltpu.sync_copy(rows_v, agg_sh.at[didx], add=True)

    plsc.subcore_barrier()
    rows = pl.ds(s * RPT, RPT)
    pltpu.sync_copy(agg_sh.at[rows], pagg_hbm.at[c, rows])


def _norm_body(deg_ref, ns_ref, nd_ref):
    dsrc = deg_ref[0, 0] + deg_ref[1, 0]
    ddst = deg_ref[0, 1] + deg_ref[1, 1]
    ns_ref[...] = lax.rsqrt(jnp.maximum(dsrc[:, 0:1], 1.0))
    nd_ref[...] = lax.rsqrt(jnp.maximum(ddst[:, 0:1], 1.0))


def _norms(deg):
    return pl.pallas_call(
        _norm_body,
        out_shape=(jax.ShapeDtypeStruct((N, 1), jnp.float32),
                   jax.ShapeDtypeStruct((N, 1), jnp.float32)),
    )(deg)


_BR = 1000


def _mm1_body(x_ref, w_ref, ns_ref, o_ref):
    o_ref[...] = jnp.dot(x_ref[...], w_ref[...],
                         preferred_element_type=jnp.float32) * ns_ref[...]


def _mm1(x, W1, ns):
    return pl.pallas_call(
        _mm1_body,
        grid=(N // _BR,),
        in_specs=[
            pl.BlockSpec((_BR, D), lambda i: (i, 0)),
            pl.BlockSpec((D, D), lambda i: (0, 0)),
            pl.BlockSpec((_BR, 1), lambda i: (i, 0)),
        ],
        out_specs=pl.BlockSpec((_BR, D), lambda i: (i, 0)),
        out_shape=jax.ShapeDtypeStruct((N, D), jnp.float32),
    )(x, W1, ns)


def _mid_body(p_ref, nd_ref, b_ref, w_ref, ns_ref, o_ref):
    h = (p_ref[0] + p_ref[1]) * nd_ref[...] + b_ref[...]
    h = jnp.maximum(h, 0.0)
    o_ref[...] = jnp.dot(h, w_ref[...],
                         preferred_element_type=jnp.float32) * ns_ref[...]


def _mid(pagg, nd, b1, W2, ns):
    return pl.pallas_call(
        _mid_body,
        grid=(N // _BR,),
        in_specs=[
            pl.BlockSpec((NC, _BR, D), lambda i: (0, i, 0)),
            pl.BlockSpec((_BR, 1), lambda i: (i, 0)),
            pl.BlockSpec((1, D), lambda i: (0, 0)),
            pl.BlockSpec((D, D), lambda i: (0, 0)),
            pl.BlockSpec((_BR, 1), lambda i: (i, 0)),
        ],
        out_specs=pl.BlockSpec((_BR, D), lambda i: (i, 0)),
        out_shape=jax.ShapeDtypeStruct((N, D), jnp.float32),
    )(pagg, nd, b1, W2, ns)


def _fin_body(p_ref, nd_ref, b_ref, wt_ref, bfc_ref, o_ref):
    h = (p_ref[0] + p_ref[1]) * nd_ref[...] + b_ref[...]
    o_ref[...] = jnp.dot(h, wt_ref[...],
                         preferred_element_type=jnp.float32) + bfc_ref[...]


def _fin(pagg, nd, b2, WfcT, bfc):
    return pl.pallas_call(
        _fin_body,
        grid=(N // _BR,),
        in_specs=[
            pl.BlockSpec((NC, _BR, D), lambda i: (0, i, 0)),
            pl.BlockSpec((_BR, 1), lambda i: (i, 0)),
            pl.BlockSpec((1, D), lambda i: (0, 0)),
            pl.BlockSpec((D, D), lambda i: (0, 0)),
            pl.BlockSpec((1, D), lambda i: (0, 0)),
        ],
        out_specs=pl.BlockSpec((_BR, D), lambda i: (i, 0)),
        out_shape=jax.ShapeDtypeStruct((N, D), jnp.float32),
    )(pagg, nd, b2, WfcT, bfc)


def kernel(in_feat, edge_index, W1, b1, W2, b2, Wfc, bfc):
    src = edge_index[0]
    dst = edge_index[1]
    deg = _deg_kernel(src, dst)
    ns, nd = _norms(deg)
    h1 = _mm1(in_feat, W1, ns)
    p1 = _agg_kernel(h1, src, dst)
    h2 = _mid(p1, nd, b1.reshape(1, D), W2, ns)
    p2 = _agg_kernel(h2, src, dst)
    out = _fin(p2, nd, b2.reshape(1, D), Wfc.T, bfc.reshape(1, D))
    return out


# speedup vs baseline: 3.8312x; 3.8312x over previous
import functools

import jax
import jax.numpy as jnp
from jax import lax
from jax.experimental import pallas as pl
from jax.experimental.pallas import tpu as pltpu
from jax.experimental.pallas import tpu_sc as plsc

N = 10000
E = 320000
D = 128

NC = 2
NS = 16
NW = NC * NS
EPC = E // NC
EPT = E // NW
NP = 10240
RPT = NP // NS
K = 80
ZR = 128

_mesh = plsc.VectorSubcoreMesh(core_axis_name="core", subcore_axis_name="subcore")


def _fill(ref, rows, value):
    cols = ref.shape[1]
    vec = jnp.full((16,), value, dtype=ref.dtype)

    @pl.loop(0, rows)
    def _(i):
        @pl.loop(0, cols, step=16)
        def _(j):
            ref[i, pl.ds(j, 16)] = vec


def _fill_rows(ref, rows, vec16):
    zvec = jnp.zeros((16,), dtype=ref.dtype)

    @pl.loop(0, rows)
    def _(i):
        for j in range(8):
            ref[i, pl.ds(j * 16, 16)] = vec16 if j == 0 else zvec


@functools.partial(
    pl.kernel,
    out_type=jax.ShapeDtypeStruct((NC, NP, D), jnp.float32),
    mesh=_mesh,
    scratch_types=[
        pltpu.VMEM_SHARED((NP, D), jnp.float32),
        pltpu.VMEM((K, D), jnp.float32),
        pltpu.VMEM((K, D), jnp.float32),
        pltpu.VMEM((ZR, D), jnp.float32),
        pltpu.VMEM((K,), jnp.int32),
        pltpu.VMEM((K,), jnp.int32),
    ],
)
def _deg_kernel(src_hbm, dst_hbm, cnt_hbm, cnt_sh, ones_s, ones_d, zeros_v,
                sidx, didx):
    c = lax.axis_index("core")
    s = lax.axis_index("subcore")
    lane = lax.broadcasted_iota(jnp.int32, (16,), 0)
    _fill_rows(ones_s, K, jnp.where(lane == 0, 1.0, 0.0).astype(jnp.float32))
    _fill_rows(ones_d, K, jnp.where(lane == 1, 1.0, 0.0).astype(jnp.float32))
    _fill(zeros_v, ZR, 0.0)

    @pl.loop(0, RPT // ZR)
    def _(z):
        pltpu.sync_copy(zeros_v, cnt_sh.at[pl.ds(s * RPT + z * ZR, ZR)])

    plsc.subcore_barrier()

    ebase = c * EPC + s * EPT

    @pl.loop(0, EPT // K)
    def _(j):
        base = ebase + j * K
        pltpu.sync_copy(src_hbm.at[pl.ds(base, K)], sidx)
        pltpu.sync_copy(dst_hbm.at[pl.ds(base, K)], didx)
        pltpu.sync_copy(ones_s, cnt_sh.at[sidx], add=True)
        pltpu.sync_copy(ones_d, cnt_sh.at[didx], add=True)

    plsc.subcore_barrier()
    rows = pl.ds(s * RPT, RPT)
    pltpu.sync_copy(cnt_sh.at[rows], cnt_hbm.at[c, rows])


@functools.partial(
    pl.kernel,
    out_type=jax.ShapeDtypeStruct((NC, NP, D), jnp.float32),
    mesh=_mesh,
    scratch_types=[
        pltpu.VMEM_SHARED((NP, D), jnp.float32),
        pltpu.VMEM((ZR, D), jnp.float32),
        pltpu.VMEM((K, D), jnp.float32),
        pltpu.VMEM((K,), jnp.int32),
        pltpu.VMEM((K,), jnp.int32),
    ],
)
def _agg_kernel(h_hbm, src_hbm, dst_hbm, pagg_hbm, agg_sh, zeros_v, rows_v,
                sidx, didx):
    c = lax.axis_index("core")
    s = lax.axis_index("subcore")
    _fill(zeros_v, ZR, 0.0)

    @pl.loop(0, RPT // ZR)
    def _(z):
        pltpu.sync_copy(zeros_v, agg_sh.at[pl.ds(s * RPT + z * ZR, ZR)])

    plsc.subcore_barrier()

    ebase = c * EPC + s * EPT

    @pl.loop(0, EPT // K)
    def _(j):
        base = ebase + j * K
        pltpu.sync_copy(src_hbm.at[pl.ds(base, K)], sidx)
        pltpu.sync_copy(dst_hbm.at[pl.ds(base, K)], didx)
        pltpu.sync_copy(h_hbm.at[sidx], rows_v)
        pltpu.sync_copy(rows_v, agg_sh.at[didx], add=True)

    plsc.subcore_barrier()
    rows = pl.ds(s * RPT, RPT)
    pltpu.sync_copy(agg_sh.at[rows], pagg_hbm.at[c, rows])


def _norm_body(deg_ref, ns_ref, nd_ref):
    cnt = deg_ref[0] + deg_ref[1]
    ns_ref[...] = lax.rsqrt(jnp.maximum(cnt[:N, 0:1], 1.0))
    nd_ref[...] = lax.rsqrt(jnp.maximum(cnt[:N, 1:2], 1.0))


def _norms(deg):
    return pl.pallas_call(
        _norm_body,
        out_shape=(jax.ShapeDtypeStruct((N, 1), jnp.float32),
                   jax.ShapeDtypeStruct((N, 1), jnp.float32)),
    )(deg)


_BR = 1000


def _mm1_body(x_ref, w_ref, ns_ref, o_ref):
    o_ref[...] = jnp.dot(x_ref[...], w_ref[...],
                         preferred_element_type=jnp.float32) * ns_ref[...]


def _mm1(x, W1, ns):
    return pl.pallas_call(
        _mm1_body,
        grid=(N // _BR,),
        in_specs=[
            pl.BlockSpec((_BR, D), lambda i: (i, 0)),
            pl.BlockSpec((D, D), lambda i: (0, 0)),
            pl.BlockSpec((_BR, 1), lambda i: (i, 0)),
        ],
        out_specs=pl.BlockSpec((_BR, D), lambda i: (i, 0)),
        out_shape=jax.ShapeDtypeStruct((N, D), jnp.float32),
    )(x, W1, ns)


def _mid_body(p_ref, nd_ref, b_ref, w_ref, ns_ref, o_ref):
    h = (p_ref[0] + p_ref[1]) * nd_ref[...] + b_ref[...]
    h = jnp.maximum(h, 0.0)
    o_ref[...] = jnp.dot(h, w_ref[...],
                         preferred_element_type=jnp.float32) * ns_ref[...]


def _mid(pagg, nd, b1, W2, ns):
    return pl.pallas_call(
        _mid_body,
        grid=(N // _BR,),
        in_specs=[
            pl.BlockSpec((NC, _BR, D), lambda i: (0, i, 0)),
            pl.BlockSpec((_BR, 1), lambda i: (i, 0)),
            pl.BlockSpec((1, D), lambda i: (0, 0)),
            pl.BlockSpec((D, D), lambda i: (0, 0)),
            pl.BlockSpec((_BR, 1), lambda i: (i, 0)),
        ],
        out_specs=pl.BlockSpec((_BR, D), lambda i: (i, 0)),
        out_shape=jax.ShapeDtypeStruct((N, D), jnp.float32),
    )(pagg, nd, b1, W2, ns)


def _fin_body(p_ref, nd_ref, b_ref, wt_ref, bfc_ref, o_ref):
    h = (p_ref[0] + p_ref[1]) * nd_ref[...] + b_ref[...]
    o_ref[...] = jnp.dot(h, wt_ref[...],
                         preferred_element_type=jnp.float32) + bfc_ref[...]


def _fin(pagg, nd, b2, WfcT, bfc):
    return pl.pallas_call(
        _fin_body,
        grid=(N // _BR,),
        in_specs=[
            pl.BlockSpec((NC, _BR, D), lambda i: (0, i, 0)),
            pl.BlockSpec((_BR, 1), lambda i: (i, 0)),
            pl.BlockSpec((1, D), lambda i: (0, 0)),
            pl.BlockSpec((D, D), lambda i: (0, 0)),
            pl.BlockSpec((1, D), lambda i: (0, 0)),
        ],
        out_specs=pl.BlockSpec((_BR, D), lambda i: (i, 0)),
        out_shape=jax.ShapeDtypeStruct((N, D), jnp.float32),
    )(pagg, nd, b2, WfcT, bfc)


def kernel(in_feat, edge_index, W1, b1, W2, b2, Wfc, bfc):
    src = edge_index[0]
    dst = edge_index[1]
    deg = _deg_kernel(src, dst)
    ns, nd = _norms(deg)
    h1 = _mm1(in_feat, W1, ns)
    p1 = _agg_kernel(h1, src, dst)
    h2 = _mid(p1, nd, b1.reshape(1, D), W2, ns)
    p2 = _agg_kernel(h2, src, dst)
    out = _fin(p2, nd, b2.reshape(1, D), Wfc.T, bfc.reshape(1, D))
    return out
